# baseline (speedup 1.0000x reference)
import functools

import jax
import jax.numpy as jnp
from jax import lax
from jax.experimental import pallas as pl
from jax.experimental.pallas import tpu as pltpu
from jax.experimental.pallas import tpu_sc as plsc


def kernel(batch, table):
    B, = batch.shape
    V, D = table.shape

    info = plsc.get_sparse_core_info()
    NC, NS = info.num_cores, info.num_subcores
    NW = NC * NS
    b_per_w = B // NW
    assert B % (8 * NW) == 0

    mesh = plsc.VectorSubcoreMesh(core_axis_name="c", subcore_axis_name="s")

    @functools.partial(
        pl.kernel,
        mesh=mesh,
        out_type=jax.ShapeDtypeStruct((B, D), jnp.float32),
        compiler_params=pltpu.CompilerParams(use_tc_tiling_on_sc=False),
        scratch_types=[
            pltpu.VMEM((b_per_w,), jnp.int32),
            pltpu.VMEM((b_per_w, D), jnp.float32),
            pltpu.SemaphoreType.DMA,
        ],
    )
    def gather_kernel(idx_hbm, table_hbm, out_hbm, idx_v, rows_v, sem):
        wid = lax.axis_index("s") * NC + lax.axis_index("c")
        base = wid * b_per_w
        pltpu.sync_copy(idx_hbm.at[pl.ds(base, b_per_w)], idx_v)
        pltpu.async_copy(table_hbm.at[idx_v], rows_v, sem).wait()
        pltpu.sync_copy(rows_v, out_hbm.at[pl.ds(base, b_per_w)])

    return gather_kernel(batch.astype(jnp.int32), table)


# speedup vs baseline: 1.7293x; 1.7293x over previous
import functools

import jax
import jax.numpy as jnp
from jax import lax
from jax.experimental import pallas as pl
from jax.experimental.pallas import tpu as pltpu
from jax.experimental.pallas import tpu_sc as plsc


def kernel(batch, table):
    B, = batch.shape
    V, D = table.shape

    info = plsc.get_sparse_core_info()
    NC, NS = info.num_cores, info.num_subcores
    NW = NC * NS
    b_per_w = B // NW
    assert B % (8 * NW) == 0

    mesh = plsc.VectorSubcoreMesh(core_axis_name="c", subcore_axis_name="s")

    @functools.partial(
        pl.kernel,
        mesh=mesh,
        out_type=jax.ShapeDtypeStruct((B, D), jnp.float32),
        scratch_types=[
            pltpu.VMEM((b_per_w,), jnp.int32),
            pltpu.VMEM((b_per_w, D), jnp.float32),
            pltpu.SemaphoreType.DMA,
        ],
    )
    def gather_kernel(idx_hbm, table_hbm, out_hbm, idx_v, rows_v, sem):
        wid = lax.axis_index("s") * NC + lax.axis_index("c")
        base = wid * b_per_w
        pltpu.sync_copy(idx_hbm.at[pl.ds(base, b_per_w)], idx_v)

        def fire(g, _):
            vec = idx_v[pl.ds(g * 16, 16)]
            for l in range(16):
                pltpu.make_async_copy(
                    table_hbm.at[pl.ds(vec[l], 1)],
                    rows_v.at[pl.ds(g * 16 + l, 1)],
                    sem,
                ).start()
            return 0

        lax.fori_loop(0, b_per_w // 16, fire, 0)
        pltpu.make_async_copy(
            table_hbm.at[pl.ds(0, b_per_w)], rows_v, sem
        ).wait()
        pltpu.sync_copy(rows_v, out_hbm.at[pl.ds(base, b_per_w)])

    return gather_kernel(batch.astype(jnp.int32), table)


# speedup vs baseline: 2.1132x; 1.2220x over previous
import functools

import jax
import jax.numpy as jnp
from jax import lax
from jax.experimental import pallas as pl
from jax.experimental.pallas import tpu as pltpu
from jax.experimental.pallas import tpu_sc as plsc

_NSLOT = 4


def kernel(batch, table):
    B, = batch.shape
    V, D = table.shape
    LANES = 128

    info = plsc.get_sparse_core_info()
    NC, NS = info.num_cores, info.num_subcores
    NW = NC * NS
    b_per_w = B // NW
    assert B % (8 * NW) == 0

    mesh = plsc.VectorSubcoreMesh(core_axis_name="c", subcore_axis_name="s")

    slab_types = [pltpu.VMEM((D, LANES), jnp.float32) for _ in range(_NSLOT)]
    sem_types = [pltpu.SemaphoreType.DMA for _ in range(_NSLOT)]

    @functools.partial(
        pl.kernel,
        mesh=mesh,
        out_type=jax.ShapeDtypeStruct((B, D), jnp.float32),
        compiler_params=pltpu.CompilerParams(needs_layout_passes=False),
        scratch_types=[
            pltpu.VMEM((b_per_w,), jnp.int32),
            pltpu.VMEM((b_per_w, D), jnp.float32),
            *slab_types,
            *sem_types,
        ],
    )
    def gather_kernel(idx_hbm, table_t_hbm, out_hbm, idx_v, rows_v, *rest):
        slabs = rest[:_NSLOT]
        sems = rest[_NSLOT:]
        wid = lax.axis_index("s") * NC + lax.axis_index("c")
        base = wid * b_per_w
        pltpu.sync_copy(idx_hbm.at[pl.ds(base, b_per_w)], idx_v)

        lane_ids = lax.iota(jnp.int32, 16)

        def fire(slot, v):
            col0 = pl.multiple_of((v // LANES) * LANES, LANES)
            pltpu.make_async_copy(
                table_t_hbm.at[:, pl.ds(col0, LANES)], slabs[slot], sems[slot]
            ).start()

        def drain(slot):
            pltpu.make_async_copy(
                table_t_hbm.at[:, pl.ds(0, LANES)], slabs[slot], sems[slot]
            ).wait()

        def extract(slot, v, i):
            j = jnp.full((16,), v % LANES, jnp.int32)
            for t in range(D // 16):
                g = plsc.load_gather(slabs[slot], [lane_ids + 16 * t, j])
                rows_v[i, pl.ds(16 * t, 16)] = g

        def chunk(c, _):
            vec = idx_v[pl.ds(c * 16, 16)]
            for w in range(16 // _NSLOT):
                for l in range(_NSLOT):
                    fire(l, vec[w * _NSLOT + l])
                for l in range(_NSLOT):
                    drain(l)
                    extract(l, vec[w * _NSLOT + l], c * 16 + w * _NSLOT + l)
            return 0

        lax.fori_loop(0, b_per_w // 16, chunk, 0)
        pltpu.sync_copy(rows_v, out_hbm.at[pl.ds(base, b_per_w)])

    return gather_kernel(batch.astype(jnp.int32), table.T)
